# Initial kernel scaffold; baseline (speedup 1.0000x reference)
import jax
import jax.numpy as jnp
from jax.experimental import pallas as pl


def kernel(indices, table):
    raise NotImplementedError("write your pallas kernel here")


# speedup vs baseline: 1.3505x; 1.3505x over previous
import functools

import jax
import jax.numpy as jnp
from jax import lax
from jax.experimental import pallas as pl
from jax.experimental.pallas import tpu as pltpu
from jax.experimental.pallas import tpu_sc as plsc

VOCAB = 1000000
EMBED = 32
BATCH = 4096
SEQ = 200

NC = 2
NS = 16
NW = NC * NS
B = BATCH * SEQ
B_PER_W = B // NW
CW = 128
NCH = B_PER_W // CW
NPAIR = NCH // 2


def _body(table_hbm, idx_hbm, out_hbm, idx_v, rows_v, sem0, sem1):
    wid = lax.axis_index("s") * NC + lax.axis_index("c")
    pltpu.sync_copy(idx_hbm.at[wid], idx_v)

    def gather(j, slot, sem):
        return pltpu.async_copy(table_hbm.at[idx_v.at[j]], rows_v.at[slot], sem)

    gather(0, 0, sem0)

    def pair(i, carry):
        j0 = 2 * i
        pltpu.make_async_copy(table_hbm.at[idx_v.at[j0]], rows_v.at[0], sem0).wait()
        gather(j0 + 1, 1, sem1)
        pltpu.sync_copy(rows_v.at[0], out_hbm.at[wid, j0])
        pltpu.make_async_copy(
            table_hbm.at[idx_v.at[j0 + 1]], rows_v.at[1], sem1
        ).wait()

        @pl.when(i + 1 < NPAIR)
        def _():
            gather(j0 + 2, 0, sem0)

        pltpu.sync_copy(rows_v.at[1], out_hbm.at[wid, j0 + 1])
        return carry

    lax.fori_loop(0, NPAIR, pair, 0)


_gather_call = functools.partial(
    pl.kernel,
    out_type=jax.ShapeDtypeStruct((NW, NCH, CW, EMBED), jnp.float32),
    mesh=plsc.VectorSubcoreMesh(core_axis_name="c", subcore_axis_name="s"),
    scratch_types=[
        pltpu.VMEM((NCH, CW), jnp.int32),
        pltpu.VMEM((2, CW, EMBED), jnp.float32),
        pltpu.SemaphoreType.DMA,
        pltpu.SemaphoreType.DMA,
    ],
    compiler_params=pltpu.CompilerParams(use_tc_tiling_on_sc=False),
)(_body)


@jax.jit
def kernel(indices, table):
    idx = jnp.reshape(indices, (NW, NCH, CW))
    out = _gather_call(table, idx)
    return jnp.reshape(out, (BATCH, SEQ, EMBED))


# speedup vs baseline: 1.4968x; 1.1083x over previous
import functools

import jax
import jax.numpy as jnp
from jax import lax
from jax.experimental import pallas as pl
from jax.experimental.pallas import tpu as pltpu
from jax.experimental.pallas import tpu_sc as plsc

VOCAB = 1000000
EMBED = 32
BATCH = 4096
SEQ = 200

NC = 2
NS = 16
NW = NC * NS
B = BATCH * SEQ
B_PER_W = B // NW
CW = 128
NCH = B_PER_W // CW
NPAIR = NCH // 2


NSLOT = 8
DEPTH = 4


def _body(table_hbm, idx_hbm, out_hbm, idx_v, rows_v, gsem, wsem):
    wid = lax.axis_index("s") * NC + lax.axis_index("c")
    pltpu.sync_copy(idx_hbm.at[wid], idx_v)

    def start_gather(j, slot):
        pltpu.async_copy(table_hbm.at[idx_v.at[j]], rows_v.at[slot], gsem.at[slot])

    def wait_gather(j, slot):
        pltpu.make_async_copy(
            table_hbm.at[idx_v.at[j]], rows_v.at[slot], gsem.at[slot]
        ).wait()

    def start_wb(j, slot):
        pltpu.async_copy(rows_v.at[slot], out_hbm.at[wid, j], wsem.at[slot])

    def wait_wb(j, slot):
        pltpu.make_async_copy(
            rows_v.at[slot], out_hbm.at[wid, j], wsem.at[slot]
        ).wait()

    for b in range(DEPTH):
        start_gather(b, b)

    def group(i, carry):
        for b in range(NSLOT):
            j = NSLOT * i + b
            wait_gather(j, b)
            start_wb(j, b)
            slot_n = (b + DEPTH) % NSLOT

            @pl.when(j >= DEPTH)
            def _():
                wait_wb(j - DEPTH, slot_n)

            @pl.when(j + DEPTH < NCH)
            def _():
                start_gather(j + DEPTH, slot_n)

        return carry

    lax.fori_loop(0, NCH // NSLOT, group, 0)

    for b in range(DEPTH, NSLOT):
        wait_wb(NCH - NSLOT + b, b)


_gather_call = functools.partial(
    pl.kernel,
    out_type=jax.ShapeDtypeStruct((NW, NCH, CW, EMBED), jnp.float32),
    mesh=plsc.VectorSubcoreMesh(core_axis_name="c", subcore_axis_name="s"),
    scratch_types=[
        pltpu.VMEM((NCH, CW), jnp.int32),
        pltpu.VMEM((NSLOT, CW, EMBED), jnp.float32),
        pltpu.SemaphoreType.DMA((NSLOT,)),
        pltpu.SemaphoreType.DMA((NSLOT,)),
    ],
    compiler_params=pltpu.CompilerParams(use_tc_tiling_on_sc=False),
)(_body)


@jax.jit
def kernel(indices, table):
    idx = jnp.reshape(indices, (NW, NCH, CW))
    out = _gather_call(table, idx)
    return jnp.reshape(out, (BATCH, SEQ, EMBED))
